# Initial kernel scaffold; baseline (speedup 1.0000x reference)
import jax
import jax.numpy as jnp
from jax.experimental import pallas as pl


def kernel(user_emb, store_emb, food_emb, us_src, us_dst, su_src, su_dst, so_src, so_dst, os_src, os_dst):
    raise NotImplementedError("write your pallas kernel here")


# speedup vs baseline: 1.0321x; 1.0321x over previous
import functools

import jax
import jax.numpy as jnp
from jax import lax
from jax.experimental import pallas as pl
from jax.experimental.pallas import tpu as pltpu
from jax.experimental.pallas import tpu_sc as plsc

N_USER = 50000
N_STORE = 10000
N_FOOD = 50000
D = 128
E = 320000

NC = 2
NS = 16
NW = NC * NS
EPW = E // NW
B = 80
NBLK = EPW // B

CHUNK_U = 6400
N_CHUNK_U = 8
CHUNK_S = 6400
N_CHUNK_S = 2
U_PAD = N_CHUNK_U * CHUNK_U
S_PAD = N_CHUNK_S * CHUNK_S
ACC_ROWS = CHUNK_U + 8
TRASH = CHUNK_U
CW = 128


def _sc_accumulate(store_emb, su_src, user_emb, us_src, food_emb, os_src,
                   su_loc, us_loc, os_loc, ones_rows, zero_rows):
  mesh = plsc.VectorSubcoreMesh(core_axis_name="c", subcore_axis_name="s")
  f32 = jnp.float32

  @functools.partial(
      pl.kernel,
      mesh=mesh,
      out_type=[
          jax.ShapeDtypeStruct((NC * U_PAD, D), f32),
          jax.ShapeDtypeStruct((NC * U_PAD, CW), f32),
          jax.ShapeDtypeStruct((NC * S_PAD, D), f32),
          jax.ShapeDtypeStruct((NC * S_PAD, CW), f32),
          jax.ShapeDtypeStruct((NC * S_PAD, D), f32),
          jax.ShapeDtypeStruct((NC * S_PAD, CW), f32),
      ],
      scratch_types=[
          pltpu.VMEM((B,), jnp.int32),
          pltpu.VMEM((B,), jnp.int32),
          pltpu.VMEM((B, D), f32),
          pltpu.VMEM((B, CW), f32),
          pltpu.VMEM_SHARED((ACC_ROWS, D), f32),
          pltpu.VMEM_SHARED((ACC_ROWS, CW), f32),
          pltpu.SemaphoreType.DMA,
      ],
  )
  def k(store_t, su_s, user_t, us_s, food_t, os_s, su_l, us_l, os_l,
        ones_h, zrow_h,
        u_sum, u_cnt, a_sum, a_cnt, b_sum, b_cnt,
        src_v, loc_v, rows_v, ones_v, acc, cacc, sem):
    cid = lax.axis_index("c")
    sid = lax.axis_index("s")
    wid = sid * NC + cid
    pltpu.sync_copy(ones_h, ones_v)

    def relation(table, src_h, loc_h, loc_off, nrows, out_sum, out_cnt,
                 out_rows, out_base):
      rpt = nrows // NS
      pltpu.sync_copy(zrow_h.at[pl.ds(0, rpt)], acc.at[pl.ds(sid * rpt, rpt)])
      pltpu.sync_copy(zrow_h.at[pl.ds(0, rpt)], cacc.at[pl.ds(sid * rpt, rpt)])
      plsc.subcore_barrier()

      def body(blk, carry):
        off = wid * EPW + blk * B
        pltpu.sync_copy(src_h.at[pl.ds(off, B)], src_v)
        pltpu.sync_copy(loc_h.at[pl.ds(loc_off + off, B)], loc_v)
        pltpu.async_copy(table.at[src_v], rows_v, sem).wait()
        pltpu.sync_copy(rows_v, acc.at[loc_v], add=True)
        pltpu.sync_copy(ones_v, cacc.at[loc_v], add=True)
        return carry

      lax.fori_loop(0, NBLK, body, 0)
      plsc.subcore_barrier()
      row0 = cid * out_rows + out_base + sid * rpt
      pltpu.sync_copy(acc.at[pl.ds(sid * rpt, rpt)],
                      out_sum.at[pl.ds(row0, rpt)])
      pltpu.sync_copy(cacc.at[pl.ds(sid * rpt, rpt)],
                      out_cnt.at[pl.ds(row0, rpt)])
      plsc.subcore_barrier()

    for cb in range(N_CHUNK_U):
      relation(store_t, su_s, su_l, cb * E, CHUNK_U, u_sum, u_cnt,
               U_PAD, cb * CHUNK_U)
    for cb in range(N_CHUNK_S):
      relation(user_t, us_s, us_l, cb * E, CHUNK_S, a_sum, a_cnt,
               S_PAD, cb * CHUNK_S)
    for cb in range(N_CHUNK_S):
      relation(food_t, os_s, os_l, cb * E, CHUNK_S, b_sum, b_cnt,
               S_PAD, cb * CHUNK_S)

  return k(store_emb, su_src, user_emb, us_src, food_emb, os_src,
           su_loc, us_loc, os_loc, ones_rows, zero_rows)


def _finalize_user(u_sum, u_cnt):
  RB = 512
  nb = U_PAD // RB

  def body(s0, s1, c0, c1, o_ref):
    s = s0[...] + s1[...]
    c = c0[:, 0:1] + c1[:, 0:1]
    o_ref[...] = s / jnp.maximum(c, 1.0)

  return pl.pallas_call(
      body,
      grid=(nb,),
      in_specs=[
          pl.BlockSpec((RB, D), lambda i: (i, 0)),
          pl.BlockSpec((RB, D), lambda i: (i + nb, 0)),
          pl.BlockSpec((RB, CW), lambda i: (i, 0)),
          pl.BlockSpec((RB, CW), lambda i: (i + nb, 0)),
      ],
      out_specs=pl.BlockSpec((RB, D), lambda i: (i, 0)),
      out_shape=jax.ShapeDtypeStruct((U_PAD, D), jnp.float32),
  )(u_sum, u_sum, u_cnt, u_cnt)


def _finalize_store(a_sum, a_cnt, b_sum, b_cnt):
  RB = 512
  nb = S_PAD // RB

  def body(as0, as1, ac0, ac1, bs0, bs1, bc0, bc1, o_ref):
    a = (as0[...] + as1[...]) / jnp.maximum(ac0[:, 0:1] + ac1[:, 0:1], 1.0)
    b = (bs0[...] + bs1[...]) / jnp.maximum(bc0[:, 0:1] + bc1[:, 0:1], 1.0)
    o_ref[...] = a + b

  return pl.pallas_call(
      body,
      grid=(nb,),
      in_specs=[
          pl.BlockSpec((RB, D), lambda i: (i, 0)),
          pl.BlockSpec((RB, D), lambda i: (i + nb, 0)),
          pl.BlockSpec((RB, CW), lambda i: (i, 0)),
          pl.BlockSpec((RB, CW), lambda i: (i + nb, 0)),
          pl.BlockSpec((RB, D), lambda i: (i, 0)),
          pl.BlockSpec((RB, D), lambda i: (i + nb, 0)),
          pl.BlockSpec((RB, CW), lambda i: (i, 0)),
          pl.BlockSpec((RB, CW), lambda i: (i + nb, 0)),
      ],
      out_specs=pl.BlockSpec((RB, D), lambda i: (i, 0)),
      out_shape=jax.ShapeDtypeStruct((S_PAD, D), jnp.float32),
  )(a_sum, a_sum, a_cnt, a_cnt, b_sum, b_sum, b_cnt, b_cnt)


def kernel(user_emb, store_emb, food_emb, us_src, us_dst, su_src, su_dst,
           so_src, so_dst, os_src, os_dst):
  def chunked_loc(dst, n_chunk, chunk):
    parts = []
    for cb in range(n_chunk):
      l = dst - cb * chunk
      parts.append(jnp.where((l >= 0) & (l < chunk), l, TRASH))
    return jnp.concatenate(parts, axis=0)

  su_loc = chunked_loc(su_dst, N_CHUNK_U, CHUNK_U)
  us_loc = chunked_loc(us_dst, N_CHUNK_S, CHUNK_S)
  os_loc = chunked_loc(os_dst, N_CHUNK_S, CHUNK_S)
  ones_rows = jnp.ones((B, CW), jnp.float32)
  zero_rows = jnp.zeros((CHUNK_U // NS, D), jnp.float32)
  u_sum, u_cnt, a_sum, a_cnt, b_sum, b_cnt = _sc_accumulate(
      store_emb, su_src, user_emb, us_src, food_emb, os_src,
      su_loc, us_loc, os_loc, ones_rows, zero_rows)
  user_h = _finalize_user(u_sum, u_cnt)[:N_USER]
  store_h = _finalize_store(a_sum, a_cnt, b_sum, b_cnt)[:N_STORE]
  return (user_h, store_h)
